# baseline (speedup 1.0000x reference)
import functools

import jax
import jax.numpy as jnp
from jax import lax
from jax.experimental import pallas as pl
from jax.experimental.pallas import tpu as pltpu
from jax.experimental.pallas import tpu_sc as plsc

BS = 64
NOBJ = 8192
K = 100
L = 16
NW = 32
ROWS_PER_W = BS // NW
NV = NOBJ // L
NBINS = 256
KPAD = 112
CANDBUF = 128
KP_IN = 34
KP_OUT = 51
KP_ROWS = BS * NOBJ * KP_IN // 128
BX_ROWS = BS * NOBJ * 4 // 128
KP_OUT_PAD = 5104
BOX_FLAT = 400


def _sc_body(logits_hbm, kp_hbm, box_hbm, ts_hbm,
             scores_hbm, kpo_hbm, boxo_hbm,
             row_v, keys_v, hist_v, cand_key_v, cand_idx_v, rank_v,
             scores_b, kp_idx_v, kp_off_v, bx_idx_v, bx_off_v,
             kp_rows, box_rows, kp_ob, box_ob, ts_v,
             sem_a, sem_b):
    cid = lax.axis_index("c")
    sid = lax.axis_index("s")
    wid = sid * 2 + cid
    iota = lax.iota(jnp.int32, L)
    ones_i = jnp.ones((L,), jnp.int32)
    zeros_i = jnp.zeros((L,), jnp.int32)
    zeros_u = jnp.zeros((L,), jnp.uint32)
    ones_f = jnp.ones((L,), jnp.float32)

    def g3(f):
        return (f >> 1) * 3 + (f & 1)

    ga = g3(iota)
    gb = g3(iota + 16)
    gc = g3(iota + 18)
    brow = iota >> 2
    bcomp = iota & 1
    bsign = jnp.where((iota & 2) == 0, jnp.float32(-0.5), jnp.float32(0.5))

    pltpu.sync_copy(ts_hbm, ts_v)

    for r in range(ROWS_PER_W):
        row = wid * ROWS_PER_W + r

        pltpu.sync_copy(logits_hbm.at[row], row_v)

        def key_body(i, _):
            x = row_v[pl.ds(i * L, L)]
            u = plsc.bitcast(x, jnp.uint32)
            sgn = u >> 31
            flip = jnp.where(sgn == jnp.uint32(1),
                             jnp.full((L,), 0xFFFFFFFF, jnp.uint32),
                             jnp.full((L,), 0x80000000, jnp.uint32))
            keys_v[pl.ds(i * L, L)] = u ^ flip
            return 0

        lax.fori_loop(0, NV, key_body, 0)

        prefix = jnp.uint32(0)
        k_rem = jnp.int32(K)
        for level in range(4):
            shift = 24 - 8 * level

            def z_body(i, _):
                hist_v[pl.ds(i * L, L)] = zeros_i
                return 0

            lax.fori_loop(0, (NBINS * L) // L, z_body, 0)

            if level == 0:
                def h_body(i, _):
                    key = keys_v[pl.ds(i * L, L)]
                    byte = ((key >> shift) & jnp.uint32(0xFF)).astype(jnp.int32)
                    plsc.addupdate_scatter(hist_v, [byte * L + iota], ones_i)
                    return 0
            else:
                hi = shift + 8
                pref_hi = prefix >> jnp.uint32(hi)

                def h_body(i, _, hi=hi, pref_hi=pref_hi, shift=shift):
                    key = keys_v[pl.ds(i * L, L)]
                    m = (key >> jnp.uint32(hi)) == pref_hi
                    byte = ((key >> shift) & jnp.uint32(0xFF)).astype(jnp.int32)
                    plsc.addupdate_scatter(hist_v, [byte * L + iota], ones_i,
                                           mask=m)
                    return 0

            lax.fori_loop(0, NV, h_body, 0)

            def g_body(g, c, k_rem=k_rem):
                found, B, n_gt, carry = c
                bins = 255 - (g * L + iota)
                acc = zeros_i
                for j in range(L):
                    acc = acc + plsc.load_gather(hist_v, [bins * L + j])
                cum = plsc.cumsum(acc)
                s_inc = cum + carry
                s_exc = s_inc - acc
                sel = s_inc >= k_rem
                sel_i = sel.astype(jnp.int32)
                first = sel & (plsc.cumsum(sel_i) == 1)
                has = jnp.sum(sel_i) > 0
                b_g = jnp.sum(jnp.where(first, bins, 0))
                n_g = jnp.sum(jnp.where(first, s_exc, 0))
                B = jnp.where(found, B, b_g)
                n_gt = jnp.where(found, n_gt, n_g)
                found = jnp.logical_or(found, has)
                carry = carry + jnp.sum(acc)
                return (found, B, n_gt, carry)

            _, B, n_gt, _ = lax.fori_loop(
                0, NBINS // L, g_body,
                (jnp.bool_(False), jnp.int32(0), jnp.int32(0), jnp.int32(0)))
            prefix = prefix | (B.astype(jnp.uint32) << jnp.uint32(shift))
            k_rem = k_rem - n_gt

        thresh = prefix
        n_eq_take = k_rem

        for z in range(CANDBUF // L):
            cand_key_v[pl.ds(z * L, L)] = zeros_u
            cand_idx_v[pl.ds(z * L, L)] = zeros_i
        for z in range(KPAD // L):
            rank_v[pl.ds(z * L, L)] = zeros_i

        def c_body(i, c):
            off, eqb = c
            key = keys_v[pl.ds(i * L, L)]
            m_gt = key > thresh
            m_eq = key == thresh
            eq_i = m_eq.astype(jnp.int32)
            eq_cum = plsc.cumsum(eq_i) + eqb
            m = m_gt | (m_eq & (eq_cum <= n_eq_take))
            plsc.store_compressed(cand_key_v.at[pl.ds(off, L)], key, mask=m)
            plsc.store_compressed(cand_idx_v.at[pl.ds(off, L)], i * L + iota,
                                  mask=m)
            off = off + jnp.sum(m.astype(jnp.int32))
            eqb = eqb + jnp.sum(eq_i)
            return (off, eqb)

        lax.fori_loop(0, NV, c_body, (jnp.int32(0), jnp.int32(0)))

        def r_body(i, _):
            ki = cand_key_v[pl.ds(i, L)][0]
            kib = jnp.full((L,), ki)
            acc = jnp.int32(0)
            for cblk in range(KPAD // L):
                kv = cand_key_v[pl.ds(cblk * L, L)]
                gt = (kv > kib).astype(jnp.int32)
                eq = ((kv == kib) & ((cblk * L + iota) < i)).astype(jnp.int32)
                acc = acc + jnp.sum(gt) + jnp.sum(eq)
            plsc.store_scatter(rank_v, [jnp.full((L,), i, jnp.int32)],
                               jnp.full((L,), acc, jnp.int32),
                               mask=iota == 0)
            return 0

        lax.fori_loop(0, K, r_body, 0)

        for cblk in range(KPAD // L):
            lanes = cblk * L + iota
            valid = lanes < K
            rk = rank_v[pl.ds(cblk * L, L)]
            key = cand_key_v[pl.ds(cblk * L, L)]
            sgn = key >> 31
            flip = jnp.where(sgn == jnp.uint32(1),
                             jnp.full((L,), 0x80000000, jnp.uint32),
                             jnp.full((L,), 0xFFFFFFFF, jnp.uint32))
            x = plsc.bitcast(key ^ flip, jnp.float32)
            sc = 1.0 / (1.0 + jnp.exp(-x))
            plsc.store_scatter(scores_b, [rk], sc, mask=valid)
            gi = row * NOBJ + cand_idx_v[pl.ds(cblk * L, L)]
            wkp = (gi * KP_IN) >> 7
            plsc.store_scatter(kp_idx_v, [jnp.full((L,), 0, jnp.int32), rk],
                               wkp, mask=valid)
            plsc.store_scatter(kp_idx_v, [jnp.full((L,), 1, jnp.int32), rk],
                               jnp.minimum(wkp + 1, KP_ROWS - 1), mask=valid)
            plsc.store_scatter(kp_off_v, [rk], (gi * KP_IN) & 127, mask=valid)
            plsc.store_scatter(bx_idx_v, [rk], (gi * 4) >> 7, mask=valid)
            plsc.store_scatter(bx_off_v, [rk], (gi * 4) & 127, mask=valid)

        pltpu.sync_copy(scores_b, scores_hbm.at[row])

        cps = [pltpu.async_copy(kp_hbm.at[kp_idx_v.at[j]],
                                kp_rows.at[pl.ds(j * K, K)], sem_a)
               for j in range(2)]
        cp_bx = pltpu.async_copy(box_hbm.at[bx_idx_v], box_rows, sem_b)
        for cp in cps:
            cp.wait()
        cp_bx.wait()

        rowf = jnp.full((L,), row, jnp.int32)
        scale_v = plsc.load_gather(ts_v, [rowf, 1 - (iota & 1)])

        def one_body(i, _):
            kp_ob[pl.ds(i * L, L)] = ones_f
            return 0

        lax.fori_loop(0, KP_OUT_PAD // L, one_body, 0)

        def kp_body(o, _):
            ob = o * KP_OUT
            offv = kp_off_v[pl.ds(o, L)][0]
            for fvec, gmap in ((iota, ga), (iota + 16, gb), (iota + 18, gc)):
                w = offv + fvec
                rr = (w >> 7) * K + o
                v = plsc.load_gather(kp_rows, [rr, w & 127])
                plsc.store_scatter(kp_ob, [ob + gmap], v * scale_v)
            return 0

        lax.fori_loop(0, K, kp_body, 0)

        scale4_v = plsc.load_gather(ts_v, [rowf, 1 - bcomp])

        def bx_body(g, _):
            rb = g * 4 + brow
            offb = plsc.load_gather(bx_off_v, [rb])
            c = plsc.load_gather(box_rows, [rb, offb + bcomp])
            s = plsc.load_gather(box_rows, [rb, offb + bcomp + 2])
            box_ob[pl.ds(g * L, L)] = (c + bsign * s) * scale4_v
            return 0

        lax.fori_loop(0, K // 4, bx_body, 0)

        pltpu.sync_copy(kp_ob, kpo_hbm.at[row])
        pltpu.sync_copy(box_ob, boxo_hbm.at[row])


_OUT_TYPE = [
    jax.ShapeDtypeStruct((BS, KPAD), jnp.float32),
    jax.ShapeDtypeStruct((BS, KP_OUT_PAD), jnp.float32),
    jax.ShapeDtypeStruct((BS, BOX_FLAT), jnp.float32),
]
_COMPILER_PARAMS = pltpu.CompilerParams(needs_layout_passes=False)
_SCRATCH = [
        pltpu.VMEM((NOBJ,), jnp.float32),
        pltpu.VMEM((NOBJ,), jnp.uint32),
        pltpu.VMEM((NBINS * L,), jnp.int32),
        pltpu.VMEM((CANDBUF,), jnp.uint32),
        pltpu.VMEM((CANDBUF,), jnp.int32),
        pltpu.VMEM((KPAD,), jnp.int32),
        pltpu.VMEM((KPAD,), jnp.float32),
        pltpu.VMEM((2, K), jnp.int32),
        pltpu.VMEM((KPAD,), jnp.int32),
        pltpu.VMEM((K,), jnp.int32),
        pltpu.VMEM((KPAD,), jnp.int32),
        pltpu.VMEM((2 * K, 128), jnp.float32),
        pltpu.VMEM((K, 128), jnp.float32),
        pltpu.VMEM((KP_OUT_PAD,), jnp.float32),
        pltpu.VMEM((BOX_FLAT,), jnp.float32),
        pltpu.VMEM((BS, 2), jnp.float32),
        pltpu.SemaphoreType.DMA,
        pltpu.SemaphoreType.DMA,
]

_sc_kernel = pl.kernel(
    _sc_body,
    out_type=_OUT_TYPE,
    mesh=plsc.VectorSubcoreMesh(core_axis_name="c", subcore_axis_name="s",
                                num_cores=2, num_subcores=16),
    compiler_params=_COMPILER_PARAMS,
    scratch_types=_SCRATCH,
)


def kernel(pred_logits, pred_keypoints, pred_boxes, target_sizes):
    kp_flat = pred_keypoints.reshape(KP_ROWS, 128)
    box_flat = pred_boxes.reshape(BX_ROWS, 128)
    scores_p, kp_p, box_p = _sc_kernel(pred_logits, kp_flat, box_flat,
                                       target_sizes)
    top_scores = scores_p[:, :K]
    top_labels = jnp.zeros((BS, K), jnp.int32)
    top_keypoints = kp_p[:, :K * KP_OUT].reshape(BS, K, 17, 3)
    top_boxes = box_p.reshape(BS, K, 4)
    return (top_scores, top_labels, top_keypoints, top_boxes)


# speedup vs baseline: 99.0313x; 99.0313x over previous
import functools

import jax
import jax.numpy as jnp
from jax import lax
from jax.experimental import pallas as pl
from jax.experimental.pallas import tpu as pltpu
from jax.experimental.pallas import tpu_sc as plsc

BS = 64
NOBJ = 8192
K = 100
L = 16
NW = 32
ROWS_PER_W = BS // NW
NV = NOBJ // L
NBINS = 256
KPAD = 112
CANDBUF = 128
KP_IN = 34
KP_OUT = 51
KP_GR = BS * NOBJ * KP_IN // 8
BX_GR = BS * NOBJ * 4 // 8
KP_OUT_PAD = 5104
BOX_FLAT = 400


def _sc_body(logits_hbm, kp_hbm, box_hbm, scale_hbm,
             scores_hbm, kpo_hbm, boxo_hbm,
             row_v, keys_v, hist_v, cand_key_v, cand_idx_v, rank_v,
             scores_b, kp_idx_v, kp_off_v, bx_idx_v,
             kp_rows, box_rows, kp_ob, box_ob, scale_buf,
             sem_a, sem_b):
    cid = lax.axis_index("c")
    sid = lax.axis_index("s")
    wid = sid * 2 + cid
    iota = lax.iota(jnp.int32, L)
    ones_i = jnp.ones((L,), jnp.int32)
    zeros_i = jnp.zeros((L,), jnp.int32)
    zeros_u = jnp.zeros((L,), jnp.uint32)
    ones_f = jnp.ones((L,), jnp.float32)

    def g3(f):
        return (f >> 1) * 3 + (f & 1)

    ga = g3(iota)
    gb = g3(iota + 16)
    gc = g3(iota + 18)
    brow = iota >> 2
    bcomp = iota & 1
    bsign = jnp.where((iota & 2) == 0, jnp.float32(-0.5), jnp.float32(0.5))

    for r in range(ROWS_PER_W):
        row = wid * ROWS_PER_W + r

        pltpu.sync_copy(logits_hbm.at[row], row_v)

        def key_body(i, _):
            x = row_v[pl.ds(i * L, L)]
            u = plsc.bitcast(x, jnp.uint32)
            sgn = u >> 31
            flip = jnp.where(sgn == jnp.uint32(1),
                             jnp.full((L,), 0xFFFFFFFF, jnp.uint32),
                             jnp.full((L,), 0x80000000, jnp.uint32))
            keys_v[pl.ds(i * L, L)] = u ^ flip
            return 0

        lax.fori_loop(0, NV, key_body, 0)

        prefix = jnp.uint32(0)
        k_rem = jnp.int32(K)
        for level in range(4):
            shift = 24 - 8 * level

            def z_body(i, _):
                hist_v[pl.ds(i * L, L)] = zeros_i
                return 0

            lax.fori_loop(0, (NBINS * L) // L, z_body, 0)

            if level == 0:
                def h_body(i, _):
                    key = keys_v[pl.ds(i * L, L)]
                    byte = ((key >> shift) & jnp.uint32(0xFF)).astype(jnp.int32)
                    plsc.addupdate_scatter(hist_v, [byte * L + iota], ones_i)
                    return 0
            else:
                hi = shift + 8
                pref_hi = prefix >> jnp.uint32(hi)

                def h_body(i, _, hi=hi, pref_hi=pref_hi, shift=shift):
                    key = keys_v[pl.ds(i * L, L)]
                    m = (key >> jnp.uint32(hi)) == pref_hi
                    byte = ((key >> shift) & jnp.uint32(0xFF)).astype(jnp.int32)
                    plsc.addupdate_scatter(hist_v, [byte * L + iota], ones_i,
                                           mask=m)
                    return 0

            lax.fori_loop(0, NV, h_body, 0)

            def g_body(g, c, k_rem=k_rem):
                found, B, n_gt, carry = c
                bins = 255 - (g * L + iota)
                acc = zeros_i
                for j in range(L):
                    acc = acc + plsc.load_gather(hist_v, [bins * L + j])
                cum = plsc.cumsum(acc)
                s_inc = cum + carry
                s_exc = s_inc - acc
                sel = s_inc >= k_rem
                sel_i = sel.astype(jnp.int32)
                first = sel & (plsc.cumsum(sel_i) == 1)
                has = jnp.sum(sel_i) > 0
                b_g = jnp.sum(jnp.where(first, bins, 0))
                n_g = jnp.sum(jnp.where(first, s_exc, 0))
                B = jnp.where(found, B, b_g)
                n_gt = jnp.where(found, n_gt, n_g)
                found = jnp.logical_or(found, has)
                carry = carry + jnp.sum(acc)
                return (found, B, n_gt, carry)

            _, B, n_gt, _ = lax.fori_loop(
                0, NBINS // L, g_body,
                (jnp.bool_(False), jnp.int32(0), jnp.int32(0), jnp.int32(0)))
            prefix = prefix | (B.astype(jnp.uint32) << jnp.uint32(shift))
            k_rem = k_rem - n_gt

        thresh = prefix
        n_eq_take = k_rem

        for z in range(CANDBUF // L):
            cand_key_v[pl.ds(z * L, L)] = zeros_u
            cand_idx_v[pl.ds(z * L, L)] = zeros_i
        for z in range(KPAD // L):
            rank_v[pl.ds(z * L, L)] = zeros_i

        def c_body(i, c):
            off, eqb = c
            key = keys_v[pl.ds(i * L, L)]
            m_gt = key > thresh
            m_eq = key == thresh
            eq_i = m_eq.astype(jnp.int32)
            eq_cum = plsc.cumsum(eq_i) + eqb
            m = m_gt | (m_eq & (eq_cum <= n_eq_take))
            plsc.store_compressed(cand_key_v.at[pl.ds(off, L)], key, mask=m)
            plsc.store_compressed(cand_idx_v.at[pl.ds(off, L)], i * L + iota,
                                  mask=m)
            off = off + jnp.sum(m.astype(jnp.int32))
            eqb = eqb + jnp.sum(eq_i)
            return (off, eqb)

        lax.fori_loop(0, NV, c_body, (jnp.int32(0), jnp.int32(0)))

        def r_body(i, _):
            ki = cand_key_v[pl.ds(i, L)][0]
            kib = jnp.full((L,), ki)
            acc = jnp.int32(0)
            for cblk in range(KPAD // L):
                kv = cand_key_v[pl.ds(cblk * L, L)]
                gt = (kv > kib).astype(jnp.int32)
                eq = ((kv == kib) & ((cblk * L + iota) < i)).astype(jnp.int32)
                acc = acc + jnp.sum(gt) + jnp.sum(eq)
            plsc.store_scatter(rank_v, [jnp.full((L,), i, jnp.int32)],
                               jnp.full((L,), acc, jnp.int32),
                               mask=iota == 0)
            return 0

        lax.fori_loop(0, K, r_body, 0)

        for cblk in range(KPAD // L):
            lanes = cblk * L + iota
            valid = lanes < K
            rk = rank_v[pl.ds(cblk * L, L)]
            key = cand_key_v[pl.ds(cblk * L, L)]
            sgn = key >> 31
            flip = jnp.where(sgn == jnp.uint32(1),
                             jnp.full((L,), 0x80000000, jnp.uint32),
                             jnp.full((L,), 0xFFFFFFFF, jnp.uint32))
            x = plsc.bitcast(key ^ flip, jnp.float32)
            sc = 1.0 / (1.0 + jnp.exp(-x))
            plsc.store_scatter(scores_b, [rk], sc, mask=valid)
            o = cand_idx_v[pl.ds(cblk * L, L)]
            tcol = ((o >> 7) * 32) + ((o & 127) >> 3)
            g_kp = row * (17 * 2048) + tcol
            for p in range(KP_IN):
                kk, cc = p >> 1, p & 1
                plsc.store_scatter(kp_idx_v,
                                   [jnp.full((L,), p, jnp.int32), rk],
                                   g_kp + (kk * 2048 + cc * 16), mask=valid)
            g_bx = row * 4096 + ((o >> 7) * 64) + ((o & 127) >> 3)
            for q in range(4):
                plsc.store_scatter(bx_idx_v,
                                   [jnp.full((L,), q, jnp.int32), rk],
                                   g_bx + q * 16, mask=valid)
            plsc.store_scatter(kp_off_v, [rk], o & 7, mask=valid)

        pltpu.sync_copy(scores_b, scores_hbm.at[row])

        cps = [pltpu.async_copy(kp_hbm.at[kp_idx_v.at[p]],
                                kp_rows.at[pl.ds(p * K, K)], sem_a)
               for p in range(KP_IN)]
        cps += [pltpu.async_copy(box_hbm.at[bx_idx_v.at[q]],
                                 box_rows.at[pl.ds(q * K, K)], sem_b)
                for q in range(4)]
        pltpu.sync_copy(scale_hbm.at[row], scale_buf)
        for cp in cps:
            cp.wait()

        scale_v = scale_buf[pl.ds(0, L)]

        def one_body(i, _):
            kp_ob[pl.ds(i * L, L)] = ones_f
            return 0

        lax.fori_loop(0, KP_OUT_PAD // L, one_body, 0)

        def kp_body(o, _):
            ob = o * KP_OUT
            offv = kp_off_v[pl.ds(o, L)][0]
            colv = jnp.full((L,), offv, jnp.int32)
            for fvec, gmap in ((iota, ga), (iota + 16, gb), (iota + 18, gc)):
                v = plsc.load_gather(kp_rows, [fvec * K + o, colv])
                plsc.store_scatter(kp_ob, [ob + gmap], v * scale_v)
            return 0

        lax.fori_loop(0, K, kp_body, 0)

        def bx_body(g, _):
            rb = g * 4 + brow
            offb = plsc.load_gather(kp_off_v, [rb])
            c = plsc.load_gather(box_rows, [bcomp * K + rb, offb])
            s = plsc.load_gather(box_rows, [(bcomp + 2) * K + rb, offb])
            box_ob[pl.ds(g * L, L)] = (c + bsign * s) * scale_v
            return 0

        lax.fori_loop(0, K // 4, bx_body, 0)

        pltpu.sync_copy(kp_ob, kpo_hbm.at[row])
        pltpu.sync_copy(box_ob, boxo_hbm.at[row])


_OUT_TYPE = [
    jax.ShapeDtypeStruct((BS, KPAD), jnp.float32),
    jax.ShapeDtypeStruct((BS, KP_OUT_PAD), jnp.float32),
    jax.ShapeDtypeStruct((BS, BOX_FLAT), jnp.float32),
]
_COMPILER_PARAMS = pltpu.CompilerParams(needs_layout_passes=False,
                                        use_tc_tiling_on_sc=False)
_SCRATCH = [
        pltpu.VMEM((NOBJ,), jnp.float32),
        pltpu.VMEM((NOBJ,), jnp.uint32),
        pltpu.VMEM((NBINS * L,), jnp.int32),
        pltpu.VMEM((CANDBUF,), jnp.uint32),
        pltpu.VMEM((CANDBUF,), jnp.int32),
        pltpu.VMEM((KPAD,), jnp.int32),
        pltpu.VMEM((KPAD,), jnp.float32),
        pltpu.VMEM((KP_IN, K), jnp.int32),
        pltpu.VMEM((KPAD,), jnp.int32),
        pltpu.VMEM((4, K), jnp.int32),
        pltpu.VMEM((KP_IN * K, 8), jnp.float32),
        pltpu.VMEM((4 * K, 8), jnp.float32),
        pltpu.VMEM((KP_OUT_PAD,), jnp.float32),
        pltpu.VMEM((BOX_FLAT,), jnp.float32),
        pltpu.VMEM((128,), jnp.float32),
        pltpu.SemaphoreType.DMA,
        pltpu.SemaphoreType.DMA,
]

_sc_kernel = pl.kernel(
    _sc_body,
    out_type=_OUT_TYPE,
    mesh=plsc.VectorSubcoreMesh(core_axis_name="c", subcore_axis_name="s",
                                num_cores=2, num_subcores=16),
    compiler_params=_COMPILER_PARAMS,
    scratch_types=_SCRATCH,
)


def kernel(pred_logits, pred_keypoints, pred_boxes, target_sizes):
    kp_flat = (pred_keypoints.reshape(BS, 64, 128, 17, 2)
               .transpose(0, 3, 1, 4, 2).reshape(KP_GR, 8))
    box_flat = (pred_boxes.reshape(BS, 64, 128, 4)
                .transpose(0, 1, 3, 2).reshape(BX_GR, 8))
    lane = jnp.arange(128)[None, :]
    scale128 = jnp.where((lane & 1) == 0, target_sizes[:, 1:2],
                         target_sizes[:, 0:1]).astype(jnp.float32)
    scores_p, kp_p, box_p = _sc_kernel(pred_logits, kp_flat, box_flat,
                                       scale128)
    top_scores = scores_p[:, :K]
    top_labels = jnp.zeros((BS, K), jnp.int32)
    top_keypoints = kp_p[:, :K * KP_OUT].reshape(BS, K, 17, 3)
    top_boxes = box_p.reshape(BS, K, 4)
    return (top_scores, top_labels, top_keypoints, top_boxes)
